# baseline (speedup 1.0000x reference)
import functools

import jax
import jax.numpy as jnp
from jax import lax
from jax.experimental import pallas as pl
from jax.experimental.pallas import tpu as pltpu
from jax.experimental.pallas import tpu_sc as plsc


def _sc_gather(emb_table, speaker_id):
    n_rows, emb_dim = emb_table.shape
    batch = speaker_id.shape[0]
    try:
        info = plsc.get_sparse_core_info()
        num_cores, num_subcores = info.num_cores, info.num_subcores
    except Exception:
        num_cores, num_subcores = 2, 16
    num_workers = num_cores * num_subcores
    b_per_w = batch // num_workers
    mesh = plsc.VectorSubcoreMesh(core_axis_name="c", subcore_axis_name="s")

    @functools.partial(
        pl.kernel,
        out_type=jax.ShapeDtypeStruct((batch, emb_dim), jnp.float32),
        mesh=mesh,
        compiler_params=pltpu.CompilerParams(use_tc_tiling_on_sc=False),
        scratch_types=[
            pltpu.VMEM((b_per_w,), jnp.int32),
            pltpu.VMEM((b_per_w, emb_dim), jnp.float32),
            pltpu.SemaphoreType.DMA,
        ],
    )
    def gather_kernel(table_hbm, idx_hbm, out_hbm, idx_v, rows_v, sem):
        wid = lax.axis_index("s") * num_cores + lax.axis_index("c")
        base = wid * b_per_w
        pltpu.sync_copy(idx_hbm.at[pl.ds(base, b_per_w)], idx_v)
        pltpu.async_copy(table_hbm.at[idx_v], rows_v, sem).wait()
        pltpu.sync_copy(rows_v, out_hbm.at[pl.ds(base, b_per_w)])

    return gather_kernel(emb_table, speaker_id)


def _concat_body(x_ref, emb_ref, o_ref):
    bb, t, d = x_ref.shape
    e = emb_ref.shape[-1]
    o_ref[:, :, :d] = x_ref[...]
    emb = emb_ref[...]
    o_ref[:, :, d:] = jnp.broadcast_to(emb[:, None, :], (bb, t, e))


def kernel(x, speaker_id, emb_table):
    b, t, d = x.shape
    e = emb_table.shape[1]
    emb = _sc_gather(emb_table, speaker_id.astype(jnp.int32))
    bb = 8
    return pl.pallas_call(
        _concat_body,
        grid=(b // bb,),
        in_specs=[
            pl.BlockSpec((bb, t, d), lambda i: (i, 0, 0)),
            pl.BlockSpec((bb, e), lambda i: (i, 0)),
        ],
        out_specs=pl.BlockSpec((bb, t, d + e), lambda i: (i, 0, 0)),
        out_shape=jax.ShapeDtypeStruct((b, t, d + e), jnp.float32),
    )(x, emb)


# speedup vs baseline: 1.1111x; 1.1111x over previous
import functools

import jax
import jax.numpy as jnp
from jax import lax
from jax.experimental import pallas as pl
from jax.experimental.pallas import tpu as pltpu
from jax.experimental.pallas import tpu_sc as plsc


def _sc_gather(emb_table, speaker_id):
    n_rows, emb_dim = emb_table.shape
    batch = speaker_id.shape[0]
    try:
        info = plsc.get_sparse_core_info()
        num_cores, num_subcores = info.num_cores, info.num_subcores
    except Exception:
        num_cores, num_subcores = 2, 16
    num_workers = num_cores * num_subcores
    b_per_w = batch // num_workers
    mesh = plsc.VectorSubcoreMesh(core_axis_name="c", subcore_axis_name="s")

    @functools.partial(
        pl.kernel,
        out_type=jax.ShapeDtypeStruct((batch, emb_dim), jnp.float32),
        mesh=mesh,
        compiler_params=pltpu.CompilerParams(use_tc_tiling_on_sc=False),
        scratch_types=[
            pltpu.VMEM((b_per_w,), jnp.int32),
            pltpu.VMEM((b_per_w, emb_dim), jnp.float32),
            pltpu.SemaphoreType.DMA,
        ],
    )
    def gather_kernel(table_hbm, idx_hbm, out_hbm, idx_v, rows_v, sem):
        wid = lax.axis_index("s") * num_cores + lax.axis_index("c")
        base = wid * b_per_w
        pltpu.sync_copy(idx_hbm.at[pl.ds(base, b_per_w)], idx_v)
        pltpu.async_copy(table_hbm.at[idx_v], rows_v, sem).wait()
        pltpu.sync_copy(rows_v, out_hbm.at[pl.ds(base, b_per_w)])

    return gather_kernel(emb_table, speaker_id)


def _concat_body(x_ref, emb_ref, o_ref):
    bb, t, d = x_ref.shape
    e = emb_ref.shape[-1]
    o_ref[:, :, :d] = x_ref[...]
    emb = emb_ref[...]
    o_ref[:, :, d:] = jnp.broadcast_to(emb[:, None, :], (bb, t, e))


def kernel(x, speaker_id, emb_table):
    b, t, d = x.shape
    e = emb_table.shape[1]
    emb = _sc_gather(emb_table, speaker_id.astype(jnp.int32))
    bb = 32
    return pl.pallas_call(
        _concat_body,
        grid=(b // bb,),
        in_specs=[
            pl.BlockSpec((bb, t, d), lambda i: (i, 0, 0)),
            pl.BlockSpec((bb, e), lambda i: (i, 0)),
        ],
        out_specs=pl.BlockSpec((bb, t, d + e), lambda i: (i, 0, 0)),
        out_shape=jax.ShapeDtypeStruct((b, t, d + e), jnp.float32),
    )(x, emb)
